# baseline (speedup 1.0000x reference)
import functools

import jax
import jax.numpy as jnp
from jax import lax
from jax.experimental import pallas as pl
from jax.experimental.pallas import tpu as pltpu
from jax.experimental.pallas import tpu_sc as plsc

N = 10000
E = 320000
NPAD = 10240
NCORE = 2
NSUB = 16
NW = NCORE * NSUB
CH = 128
K = 8
NG = 10
CPW = K * NG
EPW = CH * CPW
EPAD = NW * EPW
RPT = NPAD // NSUB
BNK = (1.0 + 1e-5) ** -0.5


def _mesh():
    return plsc.VectorSubcoreMesh(core_axis_name="c", subcore_axis_name="s")


def _make_deg_kernel():
    @functools.partial(
        pl.kernel,
        out_type=jax.ShapeDtypeStruct((NCORE, NPAD), jnp.float32),
        mesh=_mesh(),
        scratch_types=[
            pltpu.VMEM((CPW, CH), jnp.int32),
            pltpu.VMEM((CH,), jnp.float32),
            pltpu.VMEM_SHARED((NPAD,), jnp.float32),
            pltpu.SemaphoreType.DMA,
        ],
        compiler_params=pltpu.CompilerParams(use_tc_tiling_on_sc=False),
    )
    def deg_kernel(dst_hbm, zeros_hbm, out_hbm, didx, ones, acc, ssem):
        c = lax.axis_index("c")
        s = lax.axis_index("s")
        wid = c * NSUB + s
        for k in range(CH // 16):
            ones[pl.ds(k * 16, 16)] = jnp.full((16,), 1.0, jnp.float32)
        pltpu.sync_copy(dst_hbm.at[pl.ds(wid * CPW, CPW)], didx)
        pltpu.sync_copy(zeros_hbm.at[pl.ds(s * RPT, RPT)],
                        acc.at[pl.ds(s * RPT, RPT)])
        plsc.subcore_barrier()

        def drain_scatters():
            pltpu.make_async_copy(dst_hbm.at[pl.ds(0, K)],
                                  didx.at[pl.ds(0, K)], ssem).wait()

        def body(g, carry):

            @pl.when(g >= 2)
            def _():
                drain_scatters()

            for b in range(K):
                pltpu.async_copy(ones, acc.at[didx.at[g * K + b]], ssem,
                                 add=True)
            return carry

        lax.fori_loop(0, NG, body, 0)
        drain_scatters()
        drain_scatters()
        plsc.subcore_barrier()
        pltpu.sync_copy(acc.at[pl.ds(s * RPT, RPT)],
                        out_hbm.at[c, pl.ds(s * RPT, RPT)])

    return deg_kernel


def _make_agg_kernel(F, K, NG_S, NG_F):
    assert 16 * (NG_S + NG_F) * K == 2 * CPW * NSUB

    @functools.partial(
        pl.kernel,
        out_type=jax.ShapeDtypeStruct((NCORE, NPAD, F), jnp.float32),
        mesh=_mesh(),
        scratch_types=[
            pltpu.VMEM((NG_F * K, CH), jnp.int32),
            pltpu.VMEM((NG_F * K, CH), jnp.int32),
            pltpu.VMEM((3 * K * CH, F), jnp.float32),
            pltpu.VMEM_SHARED((NPAD, F), jnp.float32),
            pltpu.SemaphoreType.DMA,
            pltpu.SemaphoreType.DMA,
        ],
        compiler_params=pltpu.CompilerParams(use_tc_tiling_on_sc=False),
    )
    def agg_kernel(g_hbm, src_hbm, dst_hbm, zeros_hbm, out_hbm,
                   sidx, didx, rows, acc, gsem, ssem):
        c = lax.axis_index("c")
        s = lax.axis_index("s")
        slow = c == SLOW_CORE
        ng = jnp.where(slow, NG_S, NG_F)
        base = jnp.where(slow, s * NG_S, NSUB * NG_S + s * NG_F) * K
        pltpu.sync_copy(src_hbm.at[pl.ds(base, NG_F * K)], sidx)
        pltpu.sync_copy(dst_hbm.at[pl.ds(base, NG_F * K)], didx)
        pltpu.sync_copy(zeros_hbm.at[pl.ds(s * RPT, RPT)],
                        acc.at[pl.ds(s * RPT, RPT)])
        plsc.subcore_barrier()

        def issue_gathers(gg, sset):
            for b in range(K):
                pltpu.async_copy(
                    g_hbm.at[sidx.at[gg * K + b]],
                    rows.at[pl.ds((sset * K + b) * CH, CH)], gsem)

        def drain(sem):
            pltpu.make_async_copy(g_hbm.at[pl.ds(0, K * CH)],
                                  rows.at[pl.ds(0, K * CH)], sem).wait()

        issue_gathers(0, 0)

        def body(g, carry):

            @pl.when(g >= 2)
            def _():
                drain(ssem)

            @pl.when(g < ng - 1)
            def _():
                issue_gathers(g + 1, (g + 1) % 3)

            drain(gsem)
            for b in range(K):
                pltpu.async_copy(
                    rows.at[pl.ds(((g % 3) * K + b) * CH, CH)],
                    acc.at[didx.at[g * K + b]], ssem, add=True)
            return carry

        lax.fori_loop(0, ng, body, 0)
        drain(ssem)
        drain(ssem)
        plsc.subcore_barrier()
        pltpu.sync_copy(acc.at[pl.ds(s * RPT, RPT)],
                        out_hbm.at[c, pl.ds(s * RPT, RPT)])

    return agg_kernel


SLOW_CORE = 0

_deg_kernel = _make_deg_kernel()
_agg32 = _make_agg_kernel(32, 5, 7, 25)
_agg16 = _make_agg_kernel(16, 8, 5, 15)
_agg8 = _make_agg_kernel(8, 8, 6, 14)


def _tc_pre(xp, W1, dega, degb):
    def body(x_ref, w_ref, da_ref, db_ref, g_ref, dinv_ref):
        deg = da_ref[...] + db_ref[...] + 1.0
        dinv = lax.rsqrt(deg)
        h = jnp.dot(x_ref[...], w_ref[...], preferred_element_type=jnp.float32)
        g_ref[...] = h * dinv
        dinv_ref[...] = dinv

    return pl.pallas_call(
        body,
        out_shape=(jax.ShapeDtypeStruct((NPAD, 32), jnp.float32),
                   jax.ShapeDtypeStruct((NPAD, 1), jnp.float32)),
    )(xp, W1, dega, degb)


def _tc_mid(agg, g, dinv, b, gamma, beta, Wn):
    Fin = g.shape[1]
    Fout = Wn.shape[1]

    def body(agg_ref, g_ref, dinv_ref, b_ref, ga_ref, be_ref, w_ref, out_ref):
        ssum = agg_ref[0] + agg_ref[1]
        z = (ssum + g_ref[...]) * dinv_ref[...] + b_ref[...]
        y = jnp.maximum(z * (ga_ref[...] * BNK) + be_ref[...], 0.0)
        h = jnp.dot(y, w_ref[...], preferred_element_type=jnp.float32)
        out_ref[...] = h * dinv_ref[...]

    return pl.pallas_call(
        body,
        out_shape=jax.ShapeDtypeStruct((NPAD, Fout), jnp.float32),
    )(agg, g, dinv, b.reshape(1, Fin), gamma.reshape(1, Fin),
      beta.reshape(1, Fin), Wn)


def _tc_final(agg, g, dinv, b3p):
    def body(agg_ref, g_ref, dinv_ref, b_ref, out_ref):
        z = (agg_ref[0] + agg_ref[1] + g_ref[...]) * dinv_ref[...] + b_ref[...]
        col = lax.broadcasted_iota(jnp.int32, z.shape, 1)
        mask = col < 5
        zm = jnp.where(mask, z, -jnp.inf)
        mx = jnp.max(zm, axis=1, keepdims=True)
        e = jnp.where(mask, jnp.exp(z - mx), 0.0)
        lse = jnp.log(jnp.sum(e, axis=1, keepdims=True))
        res = z - mx - lse
        out_ref[...] = res[:, :5]

    return pl.pallas_call(
        body,
        out_shape=jax.ShapeDtypeStruct((NPAD, 5), jnp.float32),
    )(agg, g, dinv, b3p)


def kernel(x, edge_index, W1, b1, gamma1, beta1, W2, b2, gamma2, beta2, W3, b3):
    src = edge_index[0].astype(jnp.int32)
    dst = edge_index[1].astype(jnp.int32)
    npad_e = EPAD - E
    srcp = jnp.concatenate([src, jnp.zeros((npad_e,), jnp.int32)])
    pad_dst = N + jnp.arange(npad_e, dtype=jnp.int32) % (NPAD - N)
    dstp = jnp.concatenate([dst, pad_dst])
    srcp = srcp.reshape(NW * CPW, CH)
    dstp = dstp.reshape(NW * CPW, CH)
    xp = jnp.pad(x, ((0, NPAD - N), (0, 0)))

    z1 = jnp.zeros((NPAD,), jnp.float32)
    z32 = jnp.zeros((NPAD, 32), jnp.float32)
    z16 = jnp.zeros((NPAD, 16), jnp.float32)
    z8 = jnp.zeros((NPAD, 8), jnp.float32)

    degs = _deg_kernel(dstp, z1)
    g1, dinv = _tc_pre(xp, W1,
                       degs[0].reshape(NPAD, 1), degs[1].reshape(NPAD, 1))
    agg1 = _agg32(g1, srcp, dstp, z32)
    g2 = _tc_mid(agg1, g1, dinv, b1, gamma1, beta1, W2)
    agg2 = _agg16(g2, srcp, dstp, z16)
    W3p = jnp.pad(W3, ((0, 0), (0, 3)))
    g3 = _tc_mid(agg2, g2, dinv, b2, gamma2, beta2, W3p)
    agg3 = _agg8(g3, srcp, dstp, z8)
    b3p = jnp.pad(b3, (0, 3)).reshape(1, 8)
    outp = _tc_final(agg3, g3, dinv, b3p)
    return outp[:N]


# speedup vs baseline: 1.7303x; 1.7303x over previous
import functools

import jax
import jax.numpy as jnp
from jax import lax
from jax.experimental import pallas as pl
from jax.experimental.pallas import tpu as pltpu
from jax.experimental.pallas import tpu_sc as plsc

N = 10000
E = 320000
NPAD = 10240
NCORE = 2
NSUB = 16
NW = NCORE * NSUB
CH = 128
K = 8
NG = 10
CPW = K * NG
EPW = CH * CPW
EPAD = NW * EPW
RPT = NPAD // NSUB
BNK = (1.0 + 1e-5) ** -0.5


def _mesh():
    return plsc.VectorSubcoreMesh(core_axis_name="c", subcore_axis_name="s")


def _make_deg_kernel():
    @functools.partial(
        pl.kernel,
        out_type=jax.ShapeDtypeStruct((NCORE, NPAD), jnp.float32),
        mesh=_mesh(),
        scratch_types=[
            pltpu.VMEM((CPW, CH), jnp.int32),
            pltpu.VMEM((CH,), jnp.float32),
            pltpu.VMEM_SHARED((NPAD,), jnp.float32),
            pltpu.SemaphoreType.DMA,
        ],
        compiler_params=pltpu.CompilerParams(use_tc_tiling_on_sc=False),
    )
    def deg_kernel(dst_hbm, zeros_hbm, out_hbm, didx, ones, acc, ssem):
        c = lax.axis_index("c")
        s = lax.axis_index("s")
        wid = c * NSUB + s
        for k in range(CH // 16):
            ones[pl.ds(k * 16, 16)] = jnp.full((16,), 1.0, jnp.float32)
        pltpu.sync_copy(dst_hbm.at[pl.ds(wid * CPW, CPW)], didx)
        pltpu.sync_copy(zeros_hbm.at[pl.ds(s * RPT, RPT)],
                        acc.at[pl.ds(s * RPT, RPT)])
        plsc.subcore_barrier()

        def drain_scatters():
            pltpu.make_async_copy(dst_hbm.at[pl.ds(0, K)],
                                  didx.at[pl.ds(0, K)], ssem).wait()

        def body(g, carry):

            @pl.when(g >= 2)
            def _():
                drain_scatters()

            for b in range(K):
                pltpu.async_copy(ones, acc.at[didx.at[g * K + b]], ssem,
                                 add=True)
            return carry

        lax.fori_loop(0, NG, body, 0)
        drain_scatters()
        drain_scatters()
        plsc.subcore_barrier()
        pltpu.sync_copy(acc.at[pl.ds(s * RPT, RPT)],
                        out_hbm.at[c, pl.ds(s * RPT, RPT)])

    return deg_kernel


def _make_agg_kernel(F, K, NG_S, NG_F):
    assert 16 * (NG_S + NG_F) * K == 2 * CPW * NSUB

    @functools.partial(
        pl.kernel,
        out_type=jax.ShapeDtypeStruct((NCORE, NPAD, F), jnp.float32),
        mesh=_mesh(),
        scratch_types=[
            pltpu.VMEM((NG_F * K, CH), jnp.int32),
            pltpu.VMEM((NG_F * K, CH), jnp.int32),
            pltpu.VMEM((3 * K * CH, F), jnp.float32),
            pltpu.VMEM_SHARED((NPAD, F), jnp.float32),
            pltpu.SemaphoreType.DMA,
            pltpu.SemaphoreType.DMA,
        ],
        compiler_params=pltpu.CompilerParams(use_tc_tiling_on_sc=False),
    )
    def agg_kernel(g_hbm, src_hbm, dst_hbm, zeros_hbm, out_hbm,
                   sidx, didx, rows, acc, gsem, ssem):
        c = lax.axis_index("c")
        s = lax.axis_index("s")
        slow = c == SLOW_CORE
        ng = jnp.where(slow, NG_S, NG_F)
        base = jnp.where(slow, s * NG_S, NSUB * NG_S + s * NG_F) * K
        pltpu.sync_copy(src_hbm.at[pl.ds(base, NG_F * K)], sidx)
        pltpu.sync_copy(dst_hbm.at[pl.ds(base, NG_F * K)], didx)
        pltpu.sync_copy(zeros_hbm.at[pl.ds(s * RPT, RPT)],
                        acc.at[pl.ds(s * RPT, RPT)])
        plsc.subcore_barrier()

        def issue_gathers(gg, sset):
            for b in range(K):
                pltpu.async_copy(
                    g_hbm.at[sidx.at[gg * K + b]],
                    rows.at[pl.ds((sset * K + b) * CH, CH)], gsem)

        def drain(sem):
            pltpu.make_async_copy(g_hbm.at[pl.ds(0, K * CH)],
                                  rows.at[pl.ds(0, K * CH)], sem).wait()

        issue_gathers(0, 0)

        def body(g, carry):

            @pl.when(g >= 2)
            def _():
                drain(ssem)

            @pl.when(g < ng - 1)
            def _():
                issue_gathers(g + 1, (g + 1) % 3)

            drain(gsem)
            for b in range(K):
                pltpu.async_copy(
                    rows.at[pl.ds(((g % 3) * K + b) * CH, CH)],
                    acc.at[didx.at[g * K + b]], ssem, add=True)
            return carry

        lax.fori_loop(0, ng, body, 0)
        drain(ssem)
        drain(ssem)
        plsc.subcore_barrier()
        pltpu.sync_copy(acc.at[pl.ds(s * RPT, RPT)],
                        out_hbm.at[c, pl.ds(s * RPT, RPT)])

    return agg_kernel


SLOW_CORE = 0

_deg_kernel = _make_deg_kernel()
_agg32 = _make_agg_kernel(32, 5, 16, 16)
_agg16 = _make_agg_kernel(16, 8, 10, 10)
_agg8 = _make_agg_kernel(8, 8, 10, 10)


def _tc_pre(xp, W1, dega, degb):
    def body(x_ref, w_ref, da_ref, db_ref, g_ref, dinv_ref):
        deg = da_ref[...] + db_ref[...] + 1.0
        dinv = lax.rsqrt(deg)
        h = jnp.dot(x_ref[...], w_ref[...], preferred_element_type=jnp.float32)
        g_ref[...] = h * dinv
        dinv_ref[...] = dinv

    return pl.pallas_call(
        body,
        out_shape=(jax.ShapeDtypeStruct((NPAD, 32), jnp.float32),
                   jax.ShapeDtypeStruct((NPAD, 1), jnp.float32)),
    )(xp, W1, dega, degb)


def _tc_mid(agg, g, dinv, b, gamma, beta, Wn):
    Fin = g.shape[1]
    Fout = Wn.shape[1]

    def body(agg_ref, g_ref, dinv_ref, b_ref, ga_ref, be_ref, w_ref, out_ref):
        ssum = agg_ref[0] + agg_ref[1]
        z = (ssum + g_ref[...]) * dinv_ref[...] + b_ref[...]
        y = jnp.maximum(z * (ga_ref[...] * BNK) + be_ref[...], 0.0)
        h = jnp.dot(y, w_ref[...], preferred_element_type=jnp.float32)
        out_ref[...] = h * dinv_ref[...]

    return pl.pallas_call(
        body,
        out_shape=jax.ShapeDtypeStruct((NPAD, Fout), jnp.float32),
    )(agg, g, dinv, b.reshape(1, Fin), gamma.reshape(1, Fin),
      beta.reshape(1, Fin), Wn)


def _tc_final(agg, g, dinv, b3p):
    def body(agg_ref, g_ref, dinv_ref, b_ref, out_ref):
        z = (agg_ref[0] + agg_ref[1] + g_ref[...]) * dinv_ref[...] + b_ref[...]
        col = lax.broadcasted_iota(jnp.int32, z.shape, 1)
        mask = col < 5
        zm = jnp.where(mask, z, -jnp.inf)
        mx = jnp.max(zm, axis=1, keepdims=True)
        e = jnp.where(mask, jnp.exp(z - mx), 0.0)
        lse = jnp.log(jnp.sum(e, axis=1, keepdims=True))
        res = z - mx - lse
        out_ref[...] = res[:, :5]

    return pl.pallas_call(
        body,
        out_shape=jax.ShapeDtypeStruct((NPAD, 5), jnp.float32),
    )(agg, g, dinv, b3p)


def kernel(x, edge_index, W1, b1, gamma1, beta1, W2, b2, gamma2, beta2, W3, b3):
    src = edge_index[0].astype(jnp.int32)
    dst = edge_index[1].astype(jnp.int32)
    npad_e = EPAD - E
    pad_src = jnp.arange(npad_e, dtype=jnp.int32) % N
    pad_dst = N + jnp.arange(npad_e, dtype=jnp.int32) % (NPAD - N)
    srcp = jnp.concatenate([src, pad_src])
    dstp = jnp.concatenate([dst, pad_dst])
    srcp = srcp.reshape(NW * CPW, CH)
    dstp = dstp.reshape(NW * CPW, CH)
    xp = jnp.pad(x, ((0, NPAD - N), (0, 0)))

    z1 = jnp.zeros((NPAD,), jnp.float32)
    z32 = jnp.zeros((NPAD, 32), jnp.float32)
    z16 = jnp.zeros((NPAD, 16), jnp.float32)
    z8 = jnp.zeros((NPAD, 8), jnp.float32)

    degs = _deg_kernel(dstp, z1)
    g1, dinv = _tc_pre(xp, W1,
                       degs[0].reshape(NPAD, 1), degs[1].reshape(NPAD, 1))
    agg1 = _agg32(g1, srcp, dstp, z32)
    g2 = _tc_mid(agg1, g1, dinv, b1, gamma1, beta1, W2)
    agg2 = _agg16(g2, srcp, dstp, z16)
    W3p = jnp.pad(W3, ((0, 0), (0, 3)))
    g3 = _tc_mid(agg2, g2, dinv, b2, gamma2, beta2, W3p)
    agg3 = _agg8(g3, srcp, dstp, z8)
    b3p = jnp.pad(b3, (0, 3)).reshape(1, 8)
    outp = _tc_final(agg3, g3, dinv, b3p)
    return outp[:N]
